# Initial kernel scaffold; baseline (speedup 1.0000x reference)
import jax
import jax.numpy as jnp
from jax.experimental import pallas as pl


def kernel(scores, context_file_idx, context_theorem_pos, premise_file_idx, premise_end_pos, file_dep_edge_index, label_context, label_premise):
    raise NotImplementedError("write your pallas kernel here")


# speedup vs baseline: 7.1481x; 7.1481x over previous
import functools

import jax
import jax.numpy as jnp
from jax import lax
from jax.experimental import pallas as pl
from jax.experimental.pallas import tpu as pltpu

_NUM_FILES = 2048
_NEG_INF = float("-inf")


def _crows_kernel(cfi_ref, src_ref, dst_ref, out_ref, acc_ref):
    e = pl.program_id(0)
    n_e = pl.num_programs(0)

    @pl.when(e == 0)
    def _():
        acc_ref[...] = jnp.zeros_like(acc_ref)

    u = (cfi_ref[...] == src_ref[...]).astype(jnp.bfloat16)
    f_iota = lax.broadcasted_iota(jnp.int32, (dst_ref.shape[0], _NUM_FILES), 1)
    v = (dst_ref[...] == f_iota).astype(jnp.bfloat16)
    acc_ref[...] += jnp.dot(u, v, preferred_element_type=jnp.float32)

    @pl.when(e == n_e - 1)
    def _():
        out_ref[...] = (acc_ref[...] > 0.5).astype(jnp.bfloat16)


def _mask_kernel(scores_ref, cfi_ref, ct0_ref, ct1_ref, pfi_ref, pe0_ref,
                 pe1_ref, crows_ref, out_ref):
    pfi = pfi_ref[...]
    same_file = cfi_ref[...] == pfi
    pe0 = pe0_ref[...]
    pe1 = pe1_ref[...]
    ct0 = ct0_ref[...]
    ct1 = ct1_ref[...]
    before = (pe0 < ct0) | ((pe0 == ct0) & (pe1 <= ct1))
    f_iota = lax.broadcasted_iota(jnp.int32, (_NUM_FILES, pfi.shape[1]), 0)
    onehot = (f_iota == pfi).astype(jnp.bfloat16)
    imported = jnp.dot(crows_ref[...], onehot,
                       preferred_element_type=jnp.float32) > 0.5
    accessible = (same_file & before) | imported
    out_ref[...] = jnp.where(accessible, scores_ref[...], _NEG_INF)


def _rank_kernel(lc_ref, lp_ref, row_ref, out_ref):
    l = pl.program_id(0)
    j = lp_ref[l]
    row = row_ref[0]
    p_iota = lax.broadcasted_iota(jnp.int32, row.shape, 1)
    thr = jnp.max(jnp.where(p_iota == j, row, _NEG_INF))
    n_gt = jnp.sum((row > thr).astype(jnp.int32))
    n_eq_before = jnp.sum(((row == thr) & (p_iota < j)).astype(jnp.int32))
    rank = (n_gt + n_eq_before + 1).astype(jnp.float32)
    out_ref[...] = rank.reshape(1, 1, 1)


def _metrics_kernel(ranks_ref, lc_ref, lp_ref, r1_ref, r10_ref, mrr_ref,
                    n_ctx: int, n_prem: int):
    ranks = ranks_ref[...]
    lc = lc_ref[...]
    lp = lp_ref[...]
    n_lab = ranks.shape[1]
    ids = lc * n_prem + lp

    chunk = 512
    earlier_dups = jnp.zeros((1, n_lab), dtype=jnp.int32)
    l_iota = lax.broadcasted_iota(jnp.int32, (chunk, n_lab), 1)
    m_loc = lax.broadcasted_iota(jnp.int32, (chunk, n_lab), 0)
    for c in range(n_lab // chunk):
        ids_m = ids[:, c * chunk:(c + 1) * chunk].reshape(chunk, 1)
        m_glob = m_loc + c * chunk
        dup = (ids_m == ids) & (m_glob < l_iota)
        earlier_dups += jnp.sum(dup.astype(jnp.int32), axis=0, keepdims=True)
    keep = earlier_dups == 0

    i_iota = lax.broadcasted_iota(jnp.int32, (n_ctx, n_lab), 0)
    in_row = (i_iota == lc) & keep
    in_row_f = in_row.astype(jnp.float32)
    cnt = jnp.sum(in_row_f, axis=1, keepdims=True)
    hits1 = jnp.sum(in_row_f * (ranks <= 1.0), axis=1, keepdims=True)
    hits10 = jnp.sum(in_row_f * (ranks <= 10.0), axis=1, keepdims=True)
    minrank = jnp.min(jnp.where(in_row, jnp.broadcast_to(ranks, in_row.shape),
                                jnp.float32(jnp.inf)), axis=1, keepdims=True)

    valid = cnt > 0.0
    n_valid = jnp.maximum(jnp.sum(valid.astype(jnp.float32)), 1.0)
    denom = jnp.maximum(cnt, 1.0)
    r1 = jnp.sum(jnp.where(valid, hits1 / denom, 0.0)) / n_valid
    r10 = jnp.sum(jnp.where(valid, hits10 / denom, 0.0)) / n_valid
    mrr = jnp.sum(jnp.where(valid, 1.0 / minrank, 0.0)) / n_valid
    r1_ref[...] = r1.reshape(1, 1)
    r10_ref[...] = r10.reshape(1, 1)
    mrr_ref[...] = mrr.reshape(1, 1)


def kernel(scores, context_file_idx, context_theorem_pos, premise_file_idx,
           premise_end_pos, file_dep_edge_index, label_context, label_premise):
    B, P = scores.shape
    EF = file_dep_edge_index.shape[1]
    L = label_context.shape[0]
    F = _NUM_FILES

    cfi = context_file_idx.reshape(B, 1)
    ct0 = context_theorem_pos[:, 0].reshape(B, 1)
    ct1 = context_theorem_pos[:, 1].reshape(B, 1)
    pfi = premise_file_idx.reshape(1, P)
    pe0 = premise_end_pos[:, 0].reshape(1, P)
    pe1 = premise_end_pos[:, 1].reshape(1, P)
    src = file_dep_edge_index[0].reshape(1, EF)
    dst = file_dep_edge_index[1].reshape(EF, 1)
    lc = label_context.reshape(1, L).astype(jnp.int32)
    lp = label_premise.reshape(1, L).astype(jnp.int32)

    EBLK = 1024
    crows = pl.pallas_call(
        _crows_kernel,
        grid=(EF // EBLK,),
        in_specs=[
            pl.BlockSpec((B, 1), lambda e: (0, 0)),
            pl.BlockSpec((1, EBLK), lambda e: (0, e)),
            pl.BlockSpec((EBLK, 1), lambda e: (e, 0)),
        ],
        out_specs=pl.BlockSpec((B, F), lambda e: (0, 0)),
        out_shape=jax.ShapeDtypeStruct((B, F), jnp.bfloat16),
        scratch_shapes=[pltpu.VMEM((B, F), jnp.float32)],
    )(cfi, src, dst)

    PBLK = 1024
    masked = pl.pallas_call(
        _mask_kernel,
        grid=(P // PBLK,),
        in_specs=[
            pl.BlockSpec((B, PBLK), lambda p: (0, p)),
            pl.BlockSpec((B, 1), lambda p: (0, 0)),
            pl.BlockSpec((B, 1), lambda p: (0, 0)),
            pl.BlockSpec((B, 1), lambda p: (0, 0)),
            pl.BlockSpec((1, PBLK), lambda p: (0, p)),
            pl.BlockSpec((1, PBLK), lambda p: (0, p)),
            pl.BlockSpec((1, PBLK), lambda p: (0, p)),
            pl.BlockSpec((B, F), lambda p: (0, 0)),
        ],
        out_specs=pl.BlockSpec((B, PBLK), lambda p: (0, p)),
        out_shape=jax.ShapeDtypeStruct((B, P), jnp.float32),
    )(scores, cfi, ct0, ct1, pfi, pe0, pe1, crows)

    masked3 = masked.reshape(B, 1, P)
    ranks3 = pl.pallas_call(
        _rank_kernel,
        grid_spec=pltpu.PrefetchScalarGridSpec(
            num_scalar_prefetch=2,
            grid=(L,),
            in_specs=[
                pl.BlockSpec((1, 1, P), lambda l, lc_s, lp_s: (lc_s[l], 0, 0)),
            ],
            out_specs=pl.BlockSpec((1, 1, 1), lambda l, lc_s, lp_s: (l, 0, 0)),
        ),
        out_shape=jax.ShapeDtypeStruct((L, 1, 1), jnp.float32),
    )(label_context.astype(jnp.int32), label_premise.astype(jnp.int32),
      masked3)
    ranks = ranks3.reshape(1, L)

    r1, r10, mrr = pl.pallas_call(
        functools.partial(_metrics_kernel, n_ctx=B, n_prem=P),
        in_specs=[
            pl.BlockSpec((1, L), lambda: (0, 0)),
            pl.BlockSpec((1, L), lambda: (0, 0)),
            pl.BlockSpec((1, L), lambda: (0, 0)),
        ],
        out_specs=[
            pl.BlockSpec((1, 1), lambda: (0, 0)),
            pl.BlockSpec((1, 1), lambda: (0, 0)),
            pl.BlockSpec((1, 1), lambda: (0, 0)),
        ],
        out_shape=[
            jax.ShapeDtypeStruct((1, 1), jnp.float32),
            jax.ShapeDtypeStruct((1, 1), jnp.float32),
            jax.ShapeDtypeStruct((1, 1), jnp.float32),
        ],
    )(ranks, lc, lp)

    return (masked, r1[0, 0], r10[0, 0], mrr[0, 0])


# speedup vs baseline: 7.7425x; 1.0832x over previous
import functools

import jax
import jax.numpy as jnp
from jax import lax
from jax.experimental import pallas as pl
from jax.experimental.pallas import tpu as pltpu

_NUM_FILES = 2048
_NEG_INF = float("-inf")


def _crows_kernel(cfi_ref, src_ref, dst_ref, out_ref, acc_ref):
    e = pl.program_id(0)
    n_e = pl.num_programs(0)

    @pl.when(e == 0)
    def _():
        acc_ref[...] = jnp.zeros_like(acc_ref)

    u = (cfi_ref[...] == src_ref[...]).astype(jnp.bfloat16)
    f_iota = lax.broadcasted_iota(jnp.int32, (dst_ref.shape[0], _NUM_FILES), 1)
    v = (dst_ref[...] == f_iota).astype(jnp.bfloat16)
    acc_ref[...] += jnp.dot(u, v, preferred_element_type=jnp.float32)

    @pl.when(e == n_e - 1)
    def _():
        out_ref[...] = (acc_ref[...] > 0.5).astype(jnp.bfloat16)


def _mask_kernel(scores_ref, cfi_ref, ct0_ref, ct1_ref, pfi_ref, pe0_ref,
                 pe1_ref, crows_ref, out_ref):
    pfi = pfi_ref[...]
    same_file = cfi_ref[...] == pfi
    pe0 = pe0_ref[...]
    pe1 = pe1_ref[...]
    ct0 = ct0_ref[...]
    ct1 = ct1_ref[...]
    before = (pe0 < ct0) | ((pe0 == ct0) & (pe1 <= ct1))
    f_iota = lax.broadcasted_iota(jnp.int32, (_NUM_FILES, pfi.shape[1]), 0)
    onehot = (f_iota == pfi).astype(jnp.bfloat16)
    imported = jnp.dot(crows_ref[...], onehot,
                       preferred_element_type=jnp.float32) > 0.5
    accessible = (same_file & before) | imported
    out_ref[...] = jnp.where(accessible, scores_ref[...], _NEG_INF)


_RANK_GROUP = 16


def _rank_kernel(lc_ref, lp_ref, *refs):
    t = pl.program_id(0)
    row_refs = refs[:_RANK_GROUP]
    out_ref = refs[_RANK_GROUP]
    ranks = []
    for g in range(_RANK_GROUP):
        j = lp_ref[t * _RANK_GROUP + g]
        row = row_refs[g][0]
        p_iota = lax.broadcasted_iota(jnp.int32, row.shape, 1)
        thr = jnp.max(jnp.where(p_iota == j, row, _NEG_INF))
        n_gt = jnp.sum((row > thr).astype(jnp.int32))
        n_eq_before = jnp.sum(((row == thr) & (p_iota < j)).astype(jnp.int32))
        rank = (n_gt + n_eq_before + 1).astype(jnp.float32)
        ranks.append(rank.reshape(1, 1, 1))
    out_ref[...] = jnp.concatenate(ranks, axis=0)


def _metrics_kernel(ranks_ref, lc_ref, lp_ref, r1_ref, r10_ref, mrr_ref,
                    n_ctx: int, n_prem: int):
    ranks = ranks_ref[...]
    lc = lc_ref[...]
    lp = lp_ref[...]
    n_lab = ranks.shape[1]
    ids = lc * n_prem + lp

    chunk = 512
    earlier_dups = jnp.zeros((1, n_lab), dtype=jnp.int32)
    l_iota = lax.broadcasted_iota(jnp.int32, (chunk, n_lab), 1)
    m_loc = lax.broadcasted_iota(jnp.int32, (chunk, n_lab), 0)
    for c in range(n_lab // chunk):
        ids_m = ids[:, c * chunk:(c + 1) * chunk].reshape(chunk, 1)
        m_glob = m_loc + c * chunk
        dup = (ids_m == ids) & (m_glob < l_iota)
        earlier_dups += jnp.sum(dup.astype(jnp.int32), axis=0, keepdims=True)
    keep = earlier_dups == 0

    i_iota = lax.broadcasted_iota(jnp.int32, (n_ctx, n_lab), 0)
    in_row = (i_iota == lc) & keep
    in_row_f = in_row.astype(jnp.float32)
    cnt = jnp.sum(in_row_f, axis=1, keepdims=True)
    hits1 = jnp.sum(in_row_f * (ranks <= 1.0), axis=1, keepdims=True)
    hits10 = jnp.sum(in_row_f * (ranks <= 10.0), axis=1, keepdims=True)
    minrank = jnp.min(jnp.where(in_row, jnp.broadcast_to(ranks, in_row.shape),
                                jnp.float32(jnp.inf)), axis=1, keepdims=True)

    valid = cnt > 0.0
    n_valid = jnp.maximum(jnp.sum(valid.astype(jnp.float32)), 1.0)
    denom = jnp.maximum(cnt, 1.0)
    r1 = jnp.sum(jnp.where(valid, hits1 / denom, 0.0)) / n_valid
    r10 = jnp.sum(jnp.where(valid, hits10 / denom, 0.0)) / n_valid
    mrr = jnp.sum(jnp.where(valid, 1.0 / minrank, 0.0)) / n_valid
    r1_ref[...] = r1.reshape(1, 1)
    r10_ref[...] = r10.reshape(1, 1)
    mrr_ref[...] = mrr.reshape(1, 1)


def kernel(scores, context_file_idx, context_theorem_pos, premise_file_idx,
           premise_end_pos, file_dep_edge_index, label_context, label_premise):
    B, P = scores.shape
    EF = file_dep_edge_index.shape[1]
    L = label_context.shape[0]
    F = _NUM_FILES

    cfi = context_file_idx.reshape(B, 1)
    ct0 = context_theorem_pos[:, 0].reshape(B, 1)
    ct1 = context_theorem_pos[:, 1].reshape(B, 1)
    pfi = premise_file_idx.reshape(1, P)
    pe0 = premise_end_pos[:, 0].reshape(1, P)
    pe1 = premise_end_pos[:, 1].reshape(1, P)
    src = file_dep_edge_index[0].reshape(1, EF)
    dst = file_dep_edge_index[1].reshape(EF, 1)
    lc = label_context.reshape(1, L).astype(jnp.int32)
    lp = label_premise.reshape(1, L).astype(jnp.int32)

    EBLK = 1024
    crows = pl.pallas_call(
        _crows_kernel,
        grid=(EF // EBLK,),
        in_specs=[
            pl.BlockSpec((B, 1), lambda e: (0, 0)),
            pl.BlockSpec((1, EBLK), lambda e: (0, e)),
            pl.BlockSpec((EBLK, 1), lambda e: (e, 0)),
        ],
        out_specs=pl.BlockSpec((B, F), lambda e: (0, 0)),
        out_shape=jax.ShapeDtypeStruct((B, F), jnp.bfloat16),
        scratch_shapes=[pltpu.VMEM((B, F), jnp.float32)],
    )(cfi, src, dst)

    PBLK = 1024
    masked = pl.pallas_call(
        _mask_kernel,
        grid=(P // PBLK,),
        in_specs=[
            pl.BlockSpec((B, PBLK), lambda p: (0, p)),
            pl.BlockSpec((B, 1), lambda p: (0, 0)),
            pl.BlockSpec((B, 1), lambda p: (0, 0)),
            pl.BlockSpec((B, 1), lambda p: (0, 0)),
            pl.BlockSpec((1, PBLK), lambda p: (0, p)),
            pl.BlockSpec((1, PBLK), lambda p: (0, p)),
            pl.BlockSpec((1, PBLK), lambda p: (0, p)),
            pl.BlockSpec((B, F), lambda p: (0, 0)),
        ],
        out_specs=pl.BlockSpec((B, PBLK), lambda p: (0, p)),
        out_shape=jax.ShapeDtypeStruct((B, P), jnp.float32),
    )(scores, cfi, ct0, ct1, pfi, pe0, pe1, crows)

    masked3 = masked.reshape(B, 1, P)
    G = _RANK_GROUP
    row_specs = [
        pl.BlockSpec((1, 1, P),
                     lambda t, lc_s, lp_s, g=g: (lc_s[t * G + g], 0, 0))
        for g in range(G)
    ]
    ranks3 = pl.pallas_call(
        _rank_kernel,
        grid_spec=pltpu.PrefetchScalarGridSpec(
            num_scalar_prefetch=2,
            grid=(L // G,),
            in_specs=row_specs,
            out_specs=pl.BlockSpec((G, 1, 1), lambda t, lc_s, lp_s: (t, 0, 0)),
        ),
        out_shape=jax.ShapeDtypeStruct((L, 1, 1), jnp.float32),
    )(label_context.astype(jnp.int32), label_premise.astype(jnp.int32),
      *([masked3] * G))
    ranks = ranks3.reshape(1, L)

    r1, r10, mrr = pl.pallas_call(
        functools.partial(_metrics_kernel, n_ctx=B, n_prem=P),
        in_specs=[
            pl.BlockSpec((1, L), lambda: (0, 0)),
            pl.BlockSpec((1, L), lambda: (0, 0)),
            pl.BlockSpec((1, L), lambda: (0, 0)),
        ],
        out_specs=[
            pl.BlockSpec((1, 1), lambda: (0, 0)),
            pl.BlockSpec((1, 1), lambda: (0, 0)),
            pl.BlockSpec((1, 1), lambda: (0, 0)),
        ],
        out_shape=[
            jax.ShapeDtypeStruct((1, 1), jnp.float32),
            jax.ShapeDtypeStruct((1, 1), jnp.float32),
            jax.ShapeDtypeStruct((1, 1), jnp.float32),
        ],
    )(ranks, lc, lp)

    return (masked, r1[0, 0], r10[0, 0], mrr[0, 0])


# speedup vs baseline: 35.0102x; 4.5218x over previous
import functools

import jax
import jax.numpy as jnp
from jax import lax
from jax.experimental import pallas as pl
from jax.experimental.pallas import tpu as pltpu

_NUM_FILES = 2048
_NEG_INF = float("-inf")


def _crows_kernel(cfi_ref, src_ref, dst_ref, out_ref, acc_ref):
    e = pl.program_id(0)
    n_e = pl.num_programs(0)

    @pl.when(e == 0)
    def _():
        acc_ref[...] = jnp.zeros_like(acc_ref)

    u = (cfi_ref[...] == src_ref[...]).astype(jnp.bfloat16)
    f_iota = lax.broadcasted_iota(jnp.int32, (dst_ref.shape[0], _NUM_FILES), 1)
    v = (dst_ref[...] == f_iota).astype(jnp.bfloat16)
    acc_ref[...] += jnp.dot(u, v, preferred_element_type=jnp.float32)

    @pl.when(e == n_e - 1)
    def _():
        out_ref[...] = (acc_ref[...] > 0.5).astype(jnp.bfloat16)


def _mask_kernel(scores_ref, cfi_ref, ct0_ref, ct1_ref, pfi_ref, pe0_ref,
                 pe1_ref, crows_ref, out_ref):
    pfi = pfi_ref[...]
    same_file = cfi_ref[...] == pfi
    pe0 = pe0_ref[...]
    pe1 = pe1_ref[...]
    ct0 = ct0_ref[...]
    ct1 = ct1_ref[...]
    before = (pe0 < ct0) | ((pe0 == ct0) & (pe1 <= ct1))
    f_iota = lax.broadcasted_iota(jnp.int32, (_NUM_FILES, pfi.shape[1]), 0)
    onehot = (f_iota == pfi).astype(jnp.bfloat16)
    imported = jnp.dot(crows_ref[...], onehot,
                       preferred_element_type=jnp.float32) > 0.5
    accessible = (same_file & before) | imported
    out_ref[...] = jnp.where(accessible, scores_ref[...], _NEG_INF)


_RANK_GROUP = 16


def _rank_kernel(lc_ref, lp_ref, *refs):
    t = pl.program_id(0)
    row_refs = refs[:_RANK_GROUP]
    out_ref = refs[_RANK_GROUP]
    ranks = []
    for g in range(_RANK_GROUP):
        j = lp_ref[t * _RANK_GROUP + g]
        row = row_refs[g][0]
        sub = lax.broadcasted_iota(jnp.int32, row.shape, 0)
        lane = lax.broadcasted_iota(jnp.int32, row.shape, 1)
        p_iota = sub * row.shape[1] + lane
        thr = jnp.max(jnp.where(p_iota == j, row, _NEG_INF))
        n_gt = jnp.sum((row > thr).astype(jnp.int32))
        n_eq_before = jnp.sum(((row == thr) & (p_iota < j)).astype(jnp.int32))
        rank = (n_gt + n_eq_before + 1).astype(jnp.float32)
        ranks.append(rank.reshape(1, 1, 1))
    out_ref[...] = jnp.concatenate(ranks, axis=0)


def _metrics_kernel(ranks_ref, lc_ref, lp_ref, r1_ref, r10_ref, mrr_ref,
                    n_ctx: int, n_prem: int):
    ranks = ranks_ref[...]
    lc = lc_ref[...]
    lp = lp_ref[...]
    n_lab = ranks.shape[1]
    ids = lc * n_prem + lp

    chunk = 512
    earlier_dups = jnp.zeros((1, n_lab), dtype=jnp.int32)
    l_iota = lax.broadcasted_iota(jnp.int32, (chunk, n_lab), 1)
    m_loc = lax.broadcasted_iota(jnp.int32, (chunk, n_lab), 0)
    for c in range(n_lab // chunk):
        ids_m = ids[:, c * chunk:(c + 1) * chunk].reshape(chunk, 1)
        m_glob = m_loc + c * chunk
        dup = (ids_m == ids) & (m_glob < l_iota)
        earlier_dups += jnp.sum(dup.astype(jnp.int32), axis=0, keepdims=True)
    keep = earlier_dups == 0

    i_iota = lax.broadcasted_iota(jnp.int32, (n_ctx, n_lab), 0)
    in_row = (i_iota == lc) & keep
    in_row_f = in_row.astype(jnp.float32)
    cnt = jnp.sum(in_row_f, axis=1, keepdims=True)
    hits1 = jnp.sum(in_row_f * (ranks <= 1.0), axis=1, keepdims=True)
    hits10 = jnp.sum(in_row_f * (ranks <= 10.0), axis=1, keepdims=True)
    minrank = jnp.min(jnp.where(in_row, jnp.broadcast_to(ranks, in_row.shape),
                                jnp.float32(jnp.inf)), axis=1, keepdims=True)

    valid = cnt > 0.0
    n_valid = jnp.maximum(jnp.sum(valid.astype(jnp.float32)), 1.0)
    denom = jnp.maximum(cnt, 1.0)
    r1 = jnp.sum(jnp.where(valid, hits1 / denom, 0.0)) / n_valid
    r10 = jnp.sum(jnp.where(valid, hits10 / denom, 0.0)) / n_valid
    mrr = jnp.sum(jnp.where(valid, 1.0 / minrank, 0.0)) / n_valid
    r1_ref[...] = r1.reshape(1, 1)
    r10_ref[...] = r10.reshape(1, 1)
    mrr_ref[...] = mrr.reshape(1, 1)


def kernel(scores, context_file_idx, context_theorem_pos, premise_file_idx,
           premise_end_pos, file_dep_edge_index, label_context, label_premise):
    B, P = scores.shape
    EF = file_dep_edge_index.shape[1]
    L = label_context.shape[0]
    F = _NUM_FILES

    cfi = context_file_idx.reshape(B, 1)
    ct0 = context_theorem_pos[:, 0].reshape(B, 1)
    ct1 = context_theorem_pos[:, 1].reshape(B, 1)
    pfi = premise_file_idx.reshape(1, P)
    pe0 = premise_end_pos[:, 0].reshape(1, P)
    pe1 = premise_end_pos[:, 1].reshape(1, P)
    src = file_dep_edge_index[0].reshape(1, EF)
    dst = file_dep_edge_index[1].reshape(EF, 1)
    lc = label_context.reshape(1, L).astype(jnp.int32)
    lp = label_premise.reshape(1, L).astype(jnp.int32)

    EBLK = 1024
    crows = pl.pallas_call(
        _crows_kernel,
        grid=(EF // EBLK,),
        in_specs=[
            pl.BlockSpec((B, 1), lambda e: (0, 0)),
            pl.BlockSpec((1, EBLK), lambda e: (0, e)),
            pl.BlockSpec((EBLK, 1), lambda e: (e, 0)),
        ],
        out_specs=pl.BlockSpec((B, F), lambda e: (0, 0)),
        out_shape=jax.ShapeDtypeStruct((B, F), jnp.bfloat16),
        scratch_shapes=[pltpu.VMEM((B, F), jnp.float32)],
    )(cfi, src, dst)

    PBLK = 1024
    masked = pl.pallas_call(
        _mask_kernel,
        grid=(P // PBLK,),
        in_specs=[
            pl.BlockSpec((B, PBLK), lambda p: (0, p)),
            pl.BlockSpec((B, 1), lambda p: (0, 0)),
            pl.BlockSpec((B, 1), lambda p: (0, 0)),
            pl.BlockSpec((B, 1), lambda p: (0, 0)),
            pl.BlockSpec((1, PBLK), lambda p: (0, p)),
            pl.BlockSpec((1, PBLK), lambda p: (0, p)),
            pl.BlockSpec((1, PBLK), lambda p: (0, p)),
            pl.BlockSpec((B, F), lambda p: (0, 0)),
        ],
        out_specs=pl.BlockSpec((B, PBLK), lambda p: (0, p)),
        out_shape=jax.ShapeDtypeStruct((B, P), jnp.float32),
    )(scores, cfi, ct0, ct1, pfi, pe0, pe1, crows)

    masked3 = masked.reshape(B, 8, P // 8)
    G = _RANK_GROUP
    row_specs = [
        pl.BlockSpec((1, 8, P // 8),
                     lambda t, lc_s, lp_s, g=g: (lc_s[t * G + g], 0, 0))
        for g in range(G)
    ]
    ranks3 = pl.pallas_call(
        _rank_kernel,
        grid_spec=pltpu.PrefetchScalarGridSpec(
            num_scalar_prefetch=2,
            grid=(L // G,),
            in_specs=row_specs,
            out_specs=pl.BlockSpec((G, 1, 1), lambda t, lc_s, lp_s: (t, 0, 0)),
        ),
        out_shape=jax.ShapeDtypeStruct((L, 1, 1), jnp.float32),
    )(label_context.astype(jnp.int32), label_premise.astype(jnp.int32),
      *([masked3] * G))
    ranks = ranks3.reshape(1, L)

    r1, r10, mrr = pl.pallas_call(
        functools.partial(_metrics_kernel, n_ctx=B, n_prem=P),
        in_specs=[
            pl.BlockSpec((1, L), lambda: (0, 0)),
            pl.BlockSpec((1, L), lambda: (0, 0)),
            pl.BlockSpec((1, L), lambda: (0, 0)),
        ],
        out_specs=[
            pl.BlockSpec((1, 1), lambda: (0, 0)),
            pl.BlockSpec((1, 1), lambda: (0, 0)),
            pl.BlockSpec((1, 1), lambda: (0, 0)),
        ],
        out_shape=[
            jax.ShapeDtypeStruct((1, 1), jnp.float32),
            jax.ShapeDtypeStruct((1, 1), jnp.float32),
            jax.ShapeDtypeStruct((1, 1), jnp.float32),
        ],
    )(ranks, lc, lp)

    return (masked, r1[0, 0], r10[0, 0], mrr[0, 0])


# speedup vs baseline: 36.6355x; 1.0464x over previous
import functools

import jax
import jax.numpy as jnp
from jax import lax
from jax.experimental import pallas as pl
from jax.experimental.pallas import tpu as pltpu

_NUM_FILES = 2048
_NEG_INF = float("-inf")


def _crows_kernel(cfi_ref, src_ref, dst_ref, out_ref, acc_ref):
    e = pl.program_id(0)
    n_e = pl.num_programs(0)

    @pl.when(e == 0)
    def _():
        acc_ref[...] = jnp.zeros_like(acc_ref)

    u = (cfi_ref[...] == src_ref[...]).astype(jnp.bfloat16)
    f_iota = lax.broadcasted_iota(jnp.int32, (dst_ref.shape[0], _NUM_FILES), 1)
    v = (dst_ref[...] == f_iota).astype(jnp.bfloat16)
    acc_ref[...] += jnp.dot(u, v, preferred_element_type=jnp.float32)

    @pl.when(e == n_e - 1)
    def _():
        out_ref[...] = (acc_ref[...] > 0.5).astype(jnp.bfloat16)


def _mask_kernel(scores_ref, cfi_ref, ct0_ref, ct1_ref, pfi_ref, pe0_ref,
                 pe1_ref, crows_ref, out_ref):
    pfi = pfi_ref[...]
    same_file = cfi_ref[...] == pfi
    pe0 = pe0_ref[...]
    pe1 = pe1_ref[...]
    ct0 = ct0_ref[...]
    ct1 = ct1_ref[...]
    before = (pe0 < ct0) | ((pe0 == ct0) & (pe1 <= ct1))
    f_iota = lax.broadcasted_iota(jnp.int32, (_NUM_FILES, pfi.shape[1]), 0)
    onehot = (f_iota == pfi).astype(jnp.bfloat16)
    imported = jnp.dot(crows_ref[...], onehot,
                       preferred_element_type=jnp.float32) > 0.5
    accessible = (same_file & before) | imported
    out_ref[...] = jnp.where(accessible, scores_ref[...], _NEG_INF)


_RANK_GROUP = 32


def _rank_kernel(lc_ref, lp_ref, *refs):
    t = pl.program_id(0)
    row_refs = refs[:_RANK_GROUP]
    out_ref = refs[_RANK_GROUP]
    ranks = []
    for g in range(_RANK_GROUP):
        j = lp_ref[t * _RANK_GROUP + g]
        row = row_refs[g][0]
        sub = lax.broadcasted_iota(jnp.int32, row.shape, 0)
        lane = lax.broadcasted_iota(jnp.int32, row.shape, 1)
        p_iota = sub * row.shape[1] + lane
        thr = jnp.max(jnp.where(p_iota == j, row, _NEG_INF))
        above = (row > thr) | ((row == thr) & (p_iota < j))
        rank = (jnp.sum(above.astype(jnp.int32)) + 1).astype(jnp.float32)
        ranks.append(rank.reshape(1, 1, 1))
    out_ref[...] = jnp.concatenate(ranks, axis=0)


def _metrics_kernel(ranks_ref, lc_ref, lp_ref, r1_ref, r10_ref, mrr_ref,
                    n_ctx: int, n_prem: int):
    ranks = ranks_ref[...]
    lc = lc_ref[...]
    lp = lp_ref[...]
    n_lab = ranks.shape[1]
    ids = lc * n_prem + lp

    chunk = 512
    earlier_dups = jnp.zeros((1, n_lab), dtype=jnp.int32)
    l_iota = lax.broadcasted_iota(jnp.int32, (chunk, n_lab), 1)
    m_loc = lax.broadcasted_iota(jnp.int32, (chunk, n_lab), 0)
    for c in range(n_lab // chunk):
        ids_m = ids[:, c * chunk:(c + 1) * chunk].reshape(chunk, 1)
        m_glob = m_loc + c * chunk
        dup = (ids_m == ids) & (m_glob < l_iota)
        earlier_dups += jnp.sum(dup.astype(jnp.int32), axis=0, keepdims=True)
    keep = earlier_dups == 0

    i_iota = lax.broadcasted_iota(jnp.int32, (n_ctx, n_lab), 0)
    in_row = (i_iota == lc) & keep
    in_row_f = in_row.astype(jnp.float32)
    cnt = jnp.sum(in_row_f, axis=1, keepdims=True)
    hits1 = jnp.sum(in_row_f * (ranks <= 1.0), axis=1, keepdims=True)
    hits10 = jnp.sum(in_row_f * (ranks <= 10.0), axis=1, keepdims=True)
    minrank = jnp.min(jnp.where(in_row, jnp.broadcast_to(ranks, in_row.shape),
                                jnp.float32(jnp.inf)), axis=1, keepdims=True)

    valid = cnt > 0.0
    n_valid = jnp.maximum(jnp.sum(valid.astype(jnp.float32)), 1.0)
    denom = jnp.maximum(cnt, 1.0)
    r1 = jnp.sum(jnp.where(valid, hits1 / denom, 0.0)) / n_valid
    r10 = jnp.sum(jnp.where(valid, hits10 / denom, 0.0)) / n_valid
    mrr = jnp.sum(jnp.where(valid, 1.0 / minrank, 0.0)) / n_valid
    r1_ref[...] = r1.reshape(1, 1)
    r10_ref[...] = r10.reshape(1, 1)
    mrr_ref[...] = mrr.reshape(1, 1)


def kernel(scores, context_file_idx, context_theorem_pos, premise_file_idx,
           premise_end_pos, file_dep_edge_index, label_context, label_premise):
    B, P = scores.shape
    EF = file_dep_edge_index.shape[1]
    L = label_context.shape[0]
    F = _NUM_FILES

    cfi = context_file_idx.reshape(B, 1)
    ct0 = context_theorem_pos[:, 0].reshape(B, 1)
    ct1 = context_theorem_pos[:, 1].reshape(B, 1)
    pfi = premise_file_idx.reshape(1, P)
    pe0 = premise_end_pos[:, 0].reshape(1, P)
    pe1 = premise_end_pos[:, 1].reshape(1, P)
    src = file_dep_edge_index[0].reshape(1, EF)
    dst = file_dep_edge_index[1].reshape(EF, 1)
    lc = label_context.reshape(1, L).astype(jnp.int32)
    lp = label_premise.reshape(1, L).astype(jnp.int32)

    EBLK = 1024
    crows = pl.pallas_call(
        _crows_kernel,
        grid=(EF // EBLK,),
        in_specs=[
            pl.BlockSpec((B, 1), lambda e: (0, 0)),
            pl.BlockSpec((1, EBLK), lambda e: (0, e)),
            pl.BlockSpec((EBLK, 1), lambda e: (e, 0)),
        ],
        out_specs=pl.BlockSpec((B, F), lambda e: (0, 0)),
        out_shape=jax.ShapeDtypeStruct((B, F), jnp.bfloat16),
        scratch_shapes=[pltpu.VMEM((B, F), jnp.float32)],
    )(cfi, src, dst)

    PBLK = 2048
    masked = pl.pallas_call(
        _mask_kernel,
        grid=(P // PBLK,),
        in_specs=[
            pl.BlockSpec((B, PBLK), lambda p: (0, p)),
            pl.BlockSpec((B, 1), lambda p: (0, 0)),
            pl.BlockSpec((B, 1), lambda p: (0, 0)),
            pl.BlockSpec((B, 1), lambda p: (0, 0)),
            pl.BlockSpec((1, PBLK), lambda p: (0, p)),
            pl.BlockSpec((1, PBLK), lambda p: (0, p)),
            pl.BlockSpec((1, PBLK), lambda p: (0, p)),
            pl.BlockSpec((B, F), lambda p: (0, 0)),
        ],
        out_specs=pl.BlockSpec((B, PBLK), lambda p: (0, p)),
        out_shape=jax.ShapeDtypeStruct((B, P), jnp.float32),
    )(scores, cfi, ct0, ct1, pfi, pe0, pe1, crows)

    masked3 = masked.reshape(B, 8, P // 8)
    G = _RANK_GROUP
    row_specs = [
        pl.BlockSpec((1, 8, P // 8),
                     lambda t, lc_s, lp_s, g=g: (lc_s[t * G + g], 0, 0))
        for g in range(G)
    ]
    ranks3 = pl.pallas_call(
        _rank_kernel,
        grid_spec=pltpu.PrefetchScalarGridSpec(
            num_scalar_prefetch=2,
            grid=(L // G,),
            in_specs=row_specs,
            out_specs=pl.BlockSpec((G, 1, 1), lambda t, lc_s, lp_s: (t, 0, 0)),
        ),
        out_shape=jax.ShapeDtypeStruct((L, 1, 1), jnp.float32),
    )(label_context.astype(jnp.int32), label_premise.astype(jnp.int32),
      *([masked3] * G))
    ranks = ranks3.reshape(1, L)

    r1, r10, mrr = pl.pallas_call(
        functools.partial(_metrics_kernel, n_ctx=B, n_prem=P),
        in_specs=[
            pl.BlockSpec((1, L), lambda: (0, 0)),
            pl.BlockSpec((1, L), lambda: (0, 0)),
            pl.BlockSpec((1, L), lambda: (0, 0)),
        ],
        out_specs=[
            pl.BlockSpec((1, 1), lambda: (0, 0)),
            pl.BlockSpec((1, 1), lambda: (0, 0)),
            pl.BlockSpec((1, 1), lambda: (0, 0)),
        ],
        out_shape=[
            jax.ShapeDtypeStruct((1, 1), jnp.float32),
            jax.ShapeDtypeStruct((1, 1), jnp.float32),
            jax.ShapeDtypeStruct((1, 1), jnp.float32),
        ],
    )(ranks, lc, lp)

    return (masked, r1[0, 0], r10[0, 0], mrr[0, 0])


# speedup vs baseline: 38.0473x; 1.0385x over previous
import functools

import jax
import jax.numpy as jnp
from jax import lax
from jax.experimental import pallas as pl
from jax.experimental.pallas import tpu as pltpu

_NUM_FILES = 2048
_NEG_INF = float("-inf")


def _crows_kernel(cfi_ref, src_ref, dst_ref, out_ref, acc_ref):
    e = pl.program_id(0)
    n_e = pl.num_programs(0)

    @pl.when(e == 0)
    def _():
        acc_ref[...] = jnp.zeros_like(acc_ref)

    u = (cfi_ref[...] == src_ref[...]).astype(jnp.bfloat16)
    f_iota = lax.broadcasted_iota(jnp.int32, (dst_ref.shape[0], _NUM_FILES), 1)
    v = (dst_ref[...] == f_iota).astype(jnp.bfloat16)
    acc_ref[...] += jnp.dot(u, v, preferred_element_type=jnp.float32)

    @pl.when(e == n_e - 1)
    def _():
        out_ref[...] = (acc_ref[...] > 0.5).astype(jnp.bfloat16)


def _mask_kernel(scores_ref, cfi_ref, ct0_ref, ct1_ref, pfi_ref, pe0_ref,
                 pe1_ref, crows_ref, out_ref):
    pfi = pfi_ref[...]
    same_file = cfi_ref[...] == pfi
    pe0 = pe0_ref[...]
    pe1 = pe1_ref[...]
    ct0 = ct0_ref[...]
    ct1 = ct1_ref[...]
    before = (pe0 < ct0) | ((pe0 == ct0) & (pe1 <= ct1))
    f_iota = lax.broadcasted_iota(jnp.int32, (_NUM_FILES, pfi.shape[1]), 0)
    onehot = (f_iota == pfi).astype(jnp.bfloat16)
    imported = jnp.dot(crows_ref[...], onehot,
                       preferred_element_type=jnp.float32) > 0.5
    accessible = (same_file & before) | imported
    out_ref[...] = jnp.where(accessible, scores_ref[...], _NEG_INF)


_RANK_GROUP = 32


def _rank_kernel(lc_ref, lp_ref, *refs):
    t = pl.program_id(0)
    row_refs = refs[:_RANK_GROUP]
    out_ref = refs[_RANK_GROUP]
    ranks = []
    for g in range(_RANK_GROUP):
        j = lp_ref[t * _RANK_GROUP + g]
        row = row_refs[g][0]
        cw = row.shape[1]
        sub = lax.broadcasted_iota(jnp.int32, row.shape, 0)
        lane = lax.broadcasted_iota(jnp.int32, row.shape, 1)
        p_iota = sub * cw + lane
        jc = j % cw
        base = (jc // 128) * 128
        slab = row_refs[g][0, :, pl.ds(base, 128)]
        s_io = lax.broadcasted_iota(jnp.int32, (8, 128), 0)
        l_io = lax.broadcasted_iota(jnp.int32, (8, 128), 1)
        thr = jnp.max(jnp.where((s_io == j // cw) & (l_io == jc - base),
                                slab, _NEG_INF))
        above = (row > thr) | ((row == thr) & (p_iota < j))
        rank = (jnp.sum(above.astype(jnp.int32)) + 1).astype(jnp.float32)
        ranks.append(rank.reshape(1, 1, 1))
    out_ref[...] = jnp.concatenate(ranks, axis=0)


def _metrics_kernel(ranks_ref, lc_ref, lp_ref, r1_ref, r10_ref, mrr_ref,
                    n_ctx: int, n_prem: int):
    ranks = ranks_ref[...]
    lc = lc_ref[...]
    lp = lp_ref[...]
    n_lab = ranks.shape[1]
    ids = lc * n_prem + lp

    chunk = 512
    earlier_dups = jnp.zeros((1, n_lab), dtype=jnp.int32)
    l_iota = lax.broadcasted_iota(jnp.int32, (chunk, n_lab), 1)
    m_loc = lax.broadcasted_iota(jnp.int32, (chunk, n_lab), 0)
    for c in range(n_lab // chunk):
        ids_m = ids[:, c * chunk:(c + 1) * chunk].reshape(chunk, 1)
        m_glob = m_loc + c * chunk
        dup = (ids_m == ids) & (m_glob < l_iota)
        earlier_dups += jnp.sum(dup.astype(jnp.int32), axis=0, keepdims=True)
    keep = earlier_dups == 0

    i_iota = lax.broadcasted_iota(jnp.int32, (n_ctx, n_lab), 0)
    in_row = (i_iota == lc) & keep
    in_row_f = in_row.astype(jnp.float32)
    cnt = jnp.sum(in_row_f, axis=1, keepdims=True)
    hits1 = jnp.sum(in_row_f * (ranks <= 1.0), axis=1, keepdims=True)
    hits10 = jnp.sum(in_row_f * (ranks <= 10.0), axis=1, keepdims=True)
    minrank = jnp.min(jnp.where(in_row, jnp.broadcast_to(ranks, in_row.shape),
                                jnp.float32(jnp.inf)), axis=1, keepdims=True)

    valid = cnt > 0.0
    n_valid = jnp.maximum(jnp.sum(valid.astype(jnp.float32)), 1.0)
    denom = jnp.maximum(cnt, 1.0)
    r1 = jnp.sum(jnp.where(valid, hits1 / denom, 0.0)) / n_valid
    r10 = jnp.sum(jnp.where(valid, hits10 / denom, 0.0)) / n_valid
    mrr = jnp.sum(jnp.where(valid, 1.0 / minrank, 0.0)) / n_valid
    r1_ref[...] = r1.reshape(1, 1)
    r10_ref[...] = r10.reshape(1, 1)
    mrr_ref[...] = mrr.reshape(1, 1)


def kernel(scores, context_file_idx, context_theorem_pos, premise_file_idx,
           premise_end_pos, file_dep_edge_index, label_context, label_premise):
    B, P = scores.shape
    EF = file_dep_edge_index.shape[1]
    L = label_context.shape[0]
    F = _NUM_FILES

    cfi = context_file_idx.reshape(B, 1)
    ct0 = context_theorem_pos[:, 0].reshape(B, 1)
    ct1 = context_theorem_pos[:, 1].reshape(B, 1)
    pfi = premise_file_idx.reshape(1, P)
    pe0 = premise_end_pos[:, 0].reshape(1, P)
    pe1 = premise_end_pos[:, 1].reshape(1, P)
    src = file_dep_edge_index[0].reshape(1, EF)
    dst = file_dep_edge_index[1].reshape(EF, 1)
    lc = label_context.reshape(1, L).astype(jnp.int32)
    lp = label_premise.reshape(1, L).astype(jnp.int32)

    EBLK = 1024
    crows = pl.pallas_call(
        _crows_kernel,
        grid=(EF // EBLK,),
        in_specs=[
            pl.BlockSpec((B, 1), lambda e: (0, 0)),
            pl.BlockSpec((1, EBLK), lambda e: (0, e)),
            pl.BlockSpec((EBLK, 1), lambda e: (e, 0)),
        ],
        out_specs=pl.BlockSpec((B, F), lambda e: (0, 0)),
        out_shape=jax.ShapeDtypeStruct((B, F), jnp.bfloat16),
        scratch_shapes=[pltpu.VMEM((B, F), jnp.float32)],
    )(cfi, src, dst)

    PBLK = 2048
    masked = pl.pallas_call(
        _mask_kernel,
        grid=(P // PBLK,),
        in_specs=[
            pl.BlockSpec((B, PBLK), lambda p: (0, p)),
            pl.BlockSpec((B, 1), lambda p: (0, 0)),
            pl.BlockSpec((B, 1), lambda p: (0, 0)),
            pl.BlockSpec((B, 1), lambda p: (0, 0)),
            pl.BlockSpec((1, PBLK), lambda p: (0, p)),
            pl.BlockSpec((1, PBLK), lambda p: (0, p)),
            pl.BlockSpec((1, PBLK), lambda p: (0, p)),
            pl.BlockSpec((B, F), lambda p: (0, 0)),
        ],
        out_specs=pl.BlockSpec((B, PBLK), lambda p: (0, p)),
        out_shape=jax.ShapeDtypeStruct((B, P), jnp.float32),
    )(scores, cfi, ct0, ct1, pfi, pe0, pe1, crows)

    masked3 = masked.reshape(B, 8, P // 8)
    G = _RANK_GROUP
    row_specs = [
        pl.BlockSpec((1, 8, P // 8),
                     lambda t, lc_s, lp_s, g=g: (lc_s[t * G + g], 0, 0))
        for g in range(G)
    ]
    ranks3 = pl.pallas_call(
        _rank_kernel,
        grid_spec=pltpu.PrefetchScalarGridSpec(
            num_scalar_prefetch=2,
            grid=(L // G,),
            in_specs=row_specs,
            out_specs=pl.BlockSpec((G, 1, 1), lambda t, lc_s, lp_s: (t, 0, 0)),
        ),
        out_shape=jax.ShapeDtypeStruct((L, 1, 1), jnp.float32),
    )(label_context.astype(jnp.int32), label_premise.astype(jnp.int32),
      *([masked3] * G))
    ranks = ranks3.reshape(1, L)

    r1, r10, mrr = pl.pallas_call(
        functools.partial(_metrics_kernel, n_ctx=B, n_prem=P),
        in_specs=[
            pl.BlockSpec((1, L), lambda: (0, 0)),
            pl.BlockSpec((1, L), lambda: (0, 0)),
            pl.BlockSpec((1, L), lambda: (0, 0)),
        ],
        out_specs=[
            pl.BlockSpec((1, 1), lambda: (0, 0)),
            pl.BlockSpec((1, 1), lambda: (0, 0)),
            pl.BlockSpec((1, 1), lambda: (0, 0)),
        ],
        out_shape=[
            jax.ShapeDtypeStruct((1, 1), jnp.float32),
            jax.ShapeDtypeStruct((1, 1), jnp.float32),
            jax.ShapeDtypeStruct((1, 1), jnp.float32),
        ],
    )(ranks, lc, lp)

    return (masked, r1[0, 0], r10[0, 0], mrr[0, 0])


# speedup vs baseline: 38.2251x; 1.0047x over previous
import functools

import jax
import jax.numpy as jnp
from jax import lax
from jax.experimental import pallas as pl
from jax.experimental.pallas import tpu as pltpu

_NUM_FILES = 2048
_NEG_INF = float("-inf")


def _crows_kernel(cfi_ref, src_ref, dst_ref, out_ref, acc_ref):
    e = pl.program_id(0)
    n_e = pl.num_programs(0)

    @pl.when(e == 0)
    def _():
        acc_ref[...] = jnp.zeros_like(acc_ref)

    u = (cfi_ref[...] == src_ref[...]).astype(jnp.bfloat16)
    f_iota = lax.broadcasted_iota(jnp.int32, (dst_ref.shape[0], _NUM_FILES), 1)
    v = (dst_ref[...] == f_iota).astype(jnp.bfloat16)
    acc_ref[...] += jnp.dot(u, v, preferred_element_type=jnp.float32)

    @pl.when(e == n_e - 1)
    def _():
        out_ref[...] = (acc_ref[...] > 0.5).astype(jnp.bfloat16)


def _mask_kernel(scores_ref, cfi_ref, ct0_ref, ct1_ref, pfi_ref, pe0_ref,
                 pe1_ref, crows_ref, out_ref):
    pfi = pfi_ref[...]
    same_file = cfi_ref[...] == pfi
    pe0 = pe0_ref[...]
    pe1 = pe1_ref[...]
    ct0 = ct0_ref[...]
    ct1 = ct1_ref[...]
    before = (pe0 < ct0) | ((pe0 == ct0) & (pe1 <= ct1))
    f_iota = lax.broadcasted_iota(jnp.int32, (_NUM_FILES, pfi.shape[1]), 0)
    onehot = (f_iota == pfi).astype(jnp.bfloat16)
    imported = jnp.dot(crows_ref[...], onehot,
                       preferred_element_type=jnp.float32) > 0.5
    accessible = (same_file & before) | imported
    out_ref[...] = jnp.where(accessible, scores_ref[...], _NEG_INF)


_RANK_GROUP = 64


def _rank_kernel(lc_ref, lp_ref, *refs):
    t = pl.program_id(0)
    row_refs = refs[:_RANK_GROUP]
    out_ref = refs[_RANK_GROUP]
    ranks = []
    for g in range(_RANK_GROUP):
        j = lp_ref[t * _RANK_GROUP + g]
        row = row_refs[g][0]
        cw = row.shape[1]
        sub = lax.broadcasted_iota(jnp.int32, row.shape, 0)
        lane = lax.broadcasted_iota(jnp.int32, row.shape, 1)
        p_iota = sub * cw + lane
        jc = j % cw
        base = (jc // 128) * 128
        slab = row_refs[g][0, :, pl.ds(base, 128)]
        s_io = lax.broadcasted_iota(jnp.int32, (8, 128), 0)
        l_io = lax.broadcasted_iota(jnp.int32, (8, 128), 1)
        thr = jnp.max(jnp.where((s_io == j // cw) & (l_io == jc - base),
                                slab, _NEG_INF))
        above = (row > thr) | ((row == thr) & (p_iota < j))
        rank = (jnp.sum(above.astype(jnp.int32)) + 1).astype(jnp.float32)
        ranks.append(rank.reshape(1, 1, 1))
    out_ref[...] = jnp.concatenate(ranks, axis=0)


def _metrics_kernel(ranks_ref, lc_ref, lp_ref, r1_ref, r10_ref, mrr_ref,
                    n_ctx: int, n_prem: int):
    ranks = ranks_ref[...]
    lc = lc_ref[...]
    lp = lp_ref[...]
    n_lab = ranks.shape[1]
    ids = lc * n_prem + lp

    chunk = 512
    earlier_dups = jnp.zeros((1, n_lab), dtype=jnp.int32)
    l_iota = lax.broadcasted_iota(jnp.int32, (chunk, n_lab), 1)
    m_loc = lax.broadcasted_iota(jnp.int32, (chunk, n_lab), 0)
    for c in range(n_lab // chunk):
        ids_m = ids[:, c * chunk:(c + 1) * chunk].reshape(chunk, 1)
        m_glob = m_loc + c * chunk
        dup = (ids_m == ids) & (m_glob < l_iota)
        earlier_dups += jnp.sum(dup.astype(jnp.int32), axis=0, keepdims=True)
    keep = earlier_dups == 0

    i_iota = lax.broadcasted_iota(jnp.int32, (n_ctx, n_lab), 0)
    in_row = (i_iota == lc) & keep
    in_row_f = in_row.astype(jnp.float32)
    cnt = jnp.sum(in_row_f, axis=1, keepdims=True)
    hits1 = jnp.sum(in_row_f * (ranks <= 1.0), axis=1, keepdims=True)
    hits10 = jnp.sum(in_row_f * (ranks <= 10.0), axis=1, keepdims=True)
    minrank = jnp.min(jnp.where(in_row, jnp.broadcast_to(ranks, in_row.shape),
                                jnp.float32(jnp.inf)), axis=1, keepdims=True)

    valid = cnt > 0.0
    n_valid = jnp.maximum(jnp.sum(valid.astype(jnp.float32)), 1.0)
    denom = jnp.maximum(cnt, 1.0)
    r1 = jnp.sum(jnp.where(valid, hits1 / denom, 0.0)) / n_valid
    r10 = jnp.sum(jnp.where(valid, hits10 / denom, 0.0)) / n_valid
    mrr = jnp.sum(jnp.where(valid, 1.0 / minrank, 0.0)) / n_valid
    r1_ref[...] = r1.reshape(1, 1)
    r10_ref[...] = r10.reshape(1, 1)
    mrr_ref[...] = mrr.reshape(1, 1)


def kernel(scores, context_file_idx, context_theorem_pos, premise_file_idx,
           premise_end_pos, file_dep_edge_index, label_context, label_premise):
    B, P = scores.shape
    EF = file_dep_edge_index.shape[1]
    L = label_context.shape[0]
    F = _NUM_FILES

    cfi = context_file_idx.reshape(B, 1)
    ct0 = context_theorem_pos[:, 0].reshape(B, 1)
    ct1 = context_theorem_pos[:, 1].reshape(B, 1)
    pfi = premise_file_idx.reshape(1, P)
    pe0 = premise_end_pos[:, 0].reshape(1, P)
    pe1 = premise_end_pos[:, 1].reshape(1, P)
    src = file_dep_edge_index[0].reshape(1, EF)
    dst = file_dep_edge_index[1].reshape(EF, 1)
    lc = label_context.reshape(1, L).astype(jnp.int32)
    lp = label_premise.reshape(1, L).astype(jnp.int32)

    EBLK = 1024
    crows = pl.pallas_call(
        _crows_kernel,
        grid=(EF // EBLK,),
        in_specs=[
            pl.BlockSpec((B, 1), lambda e: (0, 0)),
            pl.BlockSpec((1, EBLK), lambda e: (0, e)),
            pl.BlockSpec((EBLK, 1), lambda e: (e, 0)),
        ],
        out_specs=pl.BlockSpec((B, F), lambda e: (0, 0)),
        out_shape=jax.ShapeDtypeStruct((B, F), jnp.bfloat16),
        scratch_shapes=[pltpu.VMEM((B, F), jnp.float32)],
    )(cfi, src, dst)

    PBLK = 2048
    masked = pl.pallas_call(
        _mask_kernel,
        grid=(P // PBLK,),
        in_specs=[
            pl.BlockSpec((B, PBLK), lambda p: (0, p)),
            pl.BlockSpec((B, 1), lambda p: (0, 0)),
            pl.BlockSpec((B, 1), lambda p: (0, 0)),
            pl.BlockSpec((B, 1), lambda p: (0, 0)),
            pl.BlockSpec((1, PBLK), lambda p: (0, p)),
            pl.BlockSpec((1, PBLK), lambda p: (0, p)),
            pl.BlockSpec((1, PBLK), lambda p: (0, p)),
            pl.BlockSpec((B, F), lambda p: (0, 0)),
        ],
        out_specs=pl.BlockSpec((B, PBLK), lambda p: (0, p)),
        out_shape=jax.ShapeDtypeStruct((B, P), jnp.float32),
    )(scores, cfi, ct0, ct1, pfi, pe0, pe1, crows)

    masked3 = masked.reshape(B, 8, P // 8)
    G = _RANK_GROUP
    row_specs = [
        pl.BlockSpec((1, 8, P // 8),
                     lambda t, lc_s, lp_s, g=g: (lc_s[t * G + g], 0, 0))
        for g in range(G)
    ]
    ranks3 = pl.pallas_call(
        _rank_kernel,
        grid_spec=pltpu.PrefetchScalarGridSpec(
            num_scalar_prefetch=2,
            grid=(L // G,),
            in_specs=row_specs,
            out_specs=pl.BlockSpec((G, 1, 1), lambda t, lc_s, lp_s: (t, 0, 0)),
        ),
        out_shape=jax.ShapeDtypeStruct((L, 1, 1), jnp.float32),
    )(label_context.astype(jnp.int32), label_premise.astype(jnp.int32),
      *([masked3] * G))
    ranks = ranks3.reshape(1, L)

    r1, r10, mrr = pl.pallas_call(
        functools.partial(_metrics_kernel, n_ctx=B, n_prem=P),
        in_specs=[
            pl.BlockSpec((1, L), lambda: (0, 0)),
            pl.BlockSpec((1, L), lambda: (0, 0)),
            pl.BlockSpec((1, L), lambda: (0, 0)),
        ],
        out_specs=[
            pl.BlockSpec((1, 1), lambda: (0, 0)),
            pl.BlockSpec((1, 1), lambda: (0, 0)),
            pl.BlockSpec((1, 1), lambda: (0, 0)),
        ],
        out_shape=[
            jax.ShapeDtypeStruct((1, 1), jnp.float32),
            jax.ShapeDtypeStruct((1, 1), jnp.float32),
            jax.ShapeDtypeStruct((1, 1), jnp.float32),
        ],
    )(ranks, lc, lp)

    return (masked, r1[0, 0], r10[0, 0], mrr[0, 0])
